# baseline (speedup 1.0000x reference)
import functools

import jax
import jax.numpy as jnp
from jax.experimental import pallas as pl
from jax.experimental.pallas import tpu as pltpu
from jax.experimental.pallas import tpu_sc as plsc

D_MODEL = 128
W = 128
K = 1


def _gather_rows_t(table, idx3, S, B):
    nb = B // (K * W)
    mesh = plsc.VectorSubcoreMesh(core_axis_name="core",
                                  subcore_axis_name="subcore")

    @functools.partial(
        pl.kernel,
        out_type=jax.ShapeDtypeStruct((S, B, D_MODEL), table.dtype),
        mesh=mesh,
        scratch_types=[pltpu.SemaphoreType.DMA],
    )
    def gather_kernel(table_hbm, idx_hbm, out_hbm, sem):
        def body(i_vmem, o_vmem):
            copies = [
                pltpu.async_copy(table_hbm.at[i_vmem.at[0, k]],
                                 o_vmem.at[0, pl.ds(k * W, W)], sem)
                for k in range(K)
            ]
            for c in copies:
                c.wait()

        pltpu.emit_pipeline(
            body,
            grid=(S * nb,),
            in_specs=[pl.BlockSpec((1, K, W),
                                   index_map=lambda i: (i // nb, i % nb, 0))],
            out_specs=[pl.BlockSpec((1, K * W, D_MODEL),
                                    index_map=lambda i: (i // nb, i % nb, 0))],
            core_axis_name=("core", "subcore"),
            dimension_semantics=(pltpu.PARALLEL,),
        )(idx_hbm, out_hbm)

    return gather_kernel(table, idx3)


def kernel(x, table):
    B, S = x.shape
    idx3 = x.T.astype(jnp.int32).reshape(S, B // W, W)
    out_t = _gather_rows_t(table, idx3, S, B)
    return out_t.transpose(1, 0, 2)


# speedup vs baseline: 1.2250x; 1.2250x over previous
import functools

import jax
import jax.numpy as jnp
from jax import lax
from jax.experimental import pallas as pl
from jax.experimental.pallas import tpu as pltpu
from jax.experimental.pallas import tpu_sc as plsc

D_MODEL = 128
CH = 128
NBUF = 6
NW = 32


def _gather_flat(table, idx_flat, n):
    rows_w = n // NW
    nch = rows_w // CH
    half = NBUF // 2
    mesh = plsc.VectorSubcoreMesh(core_axis_name="core",
                                  subcore_axis_name="subcore")

    @functools.partial(
        pl.kernel,
        out_type=jax.ShapeDtypeStruct((n, D_MODEL), table.dtype),
        mesh=mesh,
        scratch_types=(
            [pltpu.VMEM((rows_w,), jnp.int32)]
            + [pltpu.VMEM((CH, D_MODEL), table.dtype) for _ in range(NBUF)]
            + [pltpu.SemaphoreType.DMA for _ in range(2 * NBUF)]
        ),
    )
    def gather_kernel(table_hbm, idx_hbm, out_hbm, idx_v, *rest):
        bufs = rest[:NBUF]
        gsems = rest[NBUF:2 * NBUF]
        osems = rest[2 * NBUF:]
        wid = lax.axis_index("subcore") * 2 + lax.axis_index("core")
        base = wid * rows_w

        def gather_start(c, b):
            pltpu.async_copy(
                table_hbm.at[idx_v.at[pl.ds(c * CH, CH)]], bufs[b], gsems[b])

        def gather_wait(b):
            pltpu.make_async_copy(
                table_hbm.at[pl.ds(0, CH)], bufs[b], gsems[b]).wait()

        def store_start(c, b):
            pltpu.async_copy(
                bufs[b], out_hbm.at[pl.ds(base + c * CH, CH)], osems[b])

        def store_wait(b):
            pltpu.make_async_copy(
                bufs[b], out_hbm.at[pl.ds(base, CH)], osems[b]).wait()

        pltpu.sync_copy(idx_hbm.at[pl.ds(base, rows_w)], idx_v)
        for b in range(half):
            gather_start(b, b)

        @pl.loop(0, (nch // NBUF) * NBUF, step=NBUF)
        def _(g):
            for j in range(NBUF):
                c = g + j
                k = (j + half) % NBUF
                gather_wait(j)
                store_start(c, j)

                @pl.when(c <= nch - 1 - half)
                def _():
                    @pl.when(c >= half)
                    def _():
                        store_wait(k)
                    gather_start(c + half, k)

        for c in range((nch // NBUF) * NBUF, nch):
            gather_wait(c % NBUF)
            store_start(c, c % NBUF)
        for c in range(nch - NBUF, nch):
            store_wait(c % NBUF)

    return gather_kernel(table, idx_flat)


def kernel(x, table):
    B, S = x.shape
    n = B * S
    idx_flat = x.T.astype(jnp.int32).reshape(n)
    out2d = _gather_flat(table, idx_flat, n)
    return out2d.reshape(S, B, D_MODEL).transpose(1, 0, 2)


# speedup vs baseline: 1.2353x; 1.0084x over previous
import functools

import jax
import jax.numpy as jnp
from jax import lax
from jax.experimental import pallas as pl
from jax.experimental.pallas import tpu as pltpu
from jax.experimental.pallas import tpu_sc as plsc

D_MODEL = 128
CH = 80
NBUF = 10
NW = 32


def _gather_flat(table, idx_flat, n):
    rows_w = n // NW
    nch = rows_w // CH
    half = NBUF // 2
    mesh = plsc.VectorSubcoreMesh(core_axis_name="core",
                                  subcore_axis_name="subcore")

    @functools.partial(
        pl.kernel,
        out_type=jax.ShapeDtypeStruct((n, D_MODEL), table.dtype),
        mesh=mesh,
        scratch_types=(
            [pltpu.VMEM((rows_w,), jnp.int32)]
            + [pltpu.VMEM((CH, D_MODEL), table.dtype) for _ in range(NBUF)]
            + [pltpu.SemaphoreType.DMA for _ in range(2 * NBUF)]
        ),
    )
    def gather_kernel(table_hbm, idx_hbm, out_hbm, idx_v, *rest):
        bufs = rest[:NBUF]
        gsems = rest[NBUF:2 * NBUF]
        osems = rest[2 * NBUF:]
        wid = lax.axis_index("subcore") * 2 + lax.axis_index("core")
        base = wid * rows_w

        def gather_start(c, b):
            pltpu.async_copy(
                table_hbm.at[idx_v.at[pl.ds(c * CH, CH)]], bufs[b], gsems[b])

        def gather_wait(b):
            pltpu.make_async_copy(
                table_hbm.at[pl.ds(0, CH)], bufs[b], gsems[b]).wait()

        def store_start(c, b):
            pltpu.async_copy(
                bufs[b], out_hbm.at[pl.ds(base + c * CH, CH)], osems[b])

        def store_wait(b):
            pltpu.make_async_copy(
                bufs[b], out_hbm.at[pl.ds(base, CH)], osems[b]).wait()

        pltpu.sync_copy(idx_hbm.at[pl.ds(base, rows_w)], idx_v)
        for b in range(half):
            gather_start(b, b)

        @pl.loop(0, (nch // NBUF) * NBUF, step=NBUF)
        def _(g):
            for j in range(NBUF):
                c = g + j
                k = (j + half) % NBUF
                gather_wait(j)
                store_start(c, j)

                @pl.when(c <= nch - 1 - half)
                def _():
                    @pl.when(c >= half)
                    def _():
                        store_wait(k)
                    gather_start(c + half, k)

        for c in range((nch // NBUF) * NBUF, nch):
            gather_wait(c % NBUF)
            store_start(c, c % NBUF)
        for c in range(nch - NBUF, nch):
            store_wait(c % NBUF)

    return gather_kernel(table, idx_flat)


def kernel(x, table):
    B, S = x.shape
    n = B * S
    idx_flat = x.T.astype(jnp.int32).reshape(n)
    out2d = _gather_flat(table, idx_flat, n)
    return out2d.reshape(S, B, D_MODEL).transpose(1, 0, 2)


# speedup vs baseline: 1.2364x; 1.0009x over previous
import functools

import jax
import jax.numpy as jnp
from jax import lax
from jax.experimental import pallas as pl
from jax.experimental.pallas import tpu as pltpu
from jax.experimental.pallas import tpu_sc as plsc

D_MODEL = 128
CH = 64
NBUF = 14
NW = 32


def _gather_flat(table, idx_flat, n):
    rows_w = n // NW
    nch = rows_w // CH
    half = NBUF // 2
    mesh = plsc.VectorSubcoreMesh(core_axis_name="core",
                                  subcore_axis_name="subcore")

    @functools.partial(
        pl.kernel,
        out_type=jax.ShapeDtypeStruct((n, D_MODEL), table.dtype),
        mesh=mesh,
        scratch_types=(
            [pltpu.VMEM((rows_w,), jnp.int32)]
            + [pltpu.VMEM((CH, D_MODEL), table.dtype) for _ in range(NBUF)]
            + [pltpu.SemaphoreType.DMA for _ in range(2 * NBUF)]
        ),
    )
    def gather_kernel(table_hbm, idx_hbm, out_hbm, idx_v, *rest):
        bufs = rest[:NBUF]
        gsems = rest[NBUF:2 * NBUF]
        osems = rest[2 * NBUF:]
        wid = lax.axis_index("subcore") * 2 + lax.axis_index("core")
        base = wid * rows_w

        def gather_start(c, b):
            pltpu.async_copy(
                table_hbm.at[idx_v.at[pl.ds(c * CH, CH)]], bufs[b], gsems[b])

        def gather_wait(b):
            pltpu.make_async_copy(
                table_hbm.at[pl.ds(0, CH)], bufs[b], gsems[b]).wait()

        def store_start(c, b):
            pltpu.async_copy(
                bufs[b], out_hbm.at[pl.ds(base + c * CH, CH)], osems[b])

        def store_wait(b):
            pltpu.make_async_copy(
                bufs[b], out_hbm.at[pl.ds(base, CH)], osems[b]).wait()

        pltpu.sync_copy(idx_hbm.at[pl.ds(base, rows_w)], idx_v)
        for b in range(half):
            gather_start(b, b)

        @pl.loop(0, (nch // NBUF) * NBUF, step=NBUF)
        def _(g):
            for j in range(NBUF):
                c = g + j
                k = (j + half) % NBUF
                gather_wait(j)
                store_start(c, j)

                @pl.when(c <= nch - 1 - half)
                def _():
                    @pl.when(c >= half)
                    def _():
                        store_wait(k)
                    gather_start(c + half, k)

        for c in range((nch // NBUF) * NBUF, nch):
            gather_wait(c % NBUF)
            store_start(c, c % NBUF)
        for c in range(nch - NBUF, nch):
            store_wait(c % NBUF)

    return gather_kernel(table, idx_flat)


def kernel(x, table):
    B, S = x.shape
    n = B * S
    idx_flat = x.T.astype(jnp.int32).reshape(n)
    out2d = _gather_flat(table, idx_flat, n)
    return out2d.reshape(S, B, D_MODEL).transpose(1, 0, 2)
